# Initial kernel scaffold; baseline (speedup 1.0000x reference)
import jax
import jax.numpy as jnp
from jax.experimental import pallas as pl


def kernel(inputs, W1, g1, b1, W2, g2, b2, W3, g3, b3):
    raise NotImplementedError("write your pallas kernel here")


# speedup vs baseline: 1.0217x; 1.0217x over previous
import jax
import jax.numpy as jnp
import numpy as np
from jax.experimental import pallas as pl
from jax.experimental.pallas import tpu as pltpu

_BN_EPS = 1e-3


def _deepset_body(x_ref, w1_ref, s1_ref, b1_ref, w2_ref, s2_ref, b2_ref,
                  w3_ref, s3_ref, b3_ref, out_ref):
    x32 = x_ref[0]
    mask = jnp.any(x32 != 0.0, axis=1, keepdims=True)
    x = x32.astype(jnp.bfloat16)
    h = jnp.dot(x, w1_ref[...], preferred_element_type=jnp.float32)
    h = jnp.maximum(h * s1_ref[0] + b1_ref[0], 0.0)
    h = jnp.dot(h.astype(jnp.bfloat16), w2_ref[...],
                preferred_element_type=jnp.float32)
    h = jnp.maximum(h * s2_ref[0] + b2_ref[0], 0.0)
    h = jnp.dot(h.astype(jnp.bfloat16), w3_ref[...],
                preferred_element_type=jnp.float32)
    h = jnp.maximum(h * s3_ref[0] + b3_ref[0], 0.0)

    mf = mask.astype(jnp.float32)
    cnt = jnp.sum(mf)
    hm = h * mf
    agg_sum = jnp.sum(hm, axis=0, keepdims=True)
    agg_max = jnp.max(jnp.where(mask, h, -jnp.inf), axis=0, keepdims=True)
    ex2 = jnp.sum(hm * h, axis=0, keepdims=True) / cnt
    agg_mean = agg_sum / cnt
    var = ex2 - agg_mean * agg_mean
    agg_std = jnp.sqrt(jnp.maximum(var, 1e-12))

    H = agg_sum.shape[1]
    out_ref[0, :, 0 * H:1 * H] = agg_sum
    out_ref[0, :, 1 * H:2 * H] = agg_max
    out_ref[0, :, 2 * H:3 * H] = agg_mean
    out_ref[0, :, 3 * H:4 * H] = agg_std


def kernel(inputs, W1, g1, b1, W2, g2, b2, W3, g3, b3):
    B, L, F = inputs.shape
    H = W3.shape[1]
    inv = np.float32(1.0 / np.sqrt(1.0 + _BN_EPS))
    s1 = (g1 * inv).reshape(1, -1)
    s2 = (g2 * inv).reshape(1, -1)
    s3 = (g3 * inv).reshape(1, -1)
    b1 = b1.reshape(1, -1)
    b2 = b2.reshape(1, -1)
    b3 = b3.reshape(1, -1)
    w1 = W1.astype(jnp.bfloat16)
    w2 = W2.astype(jnp.bfloat16)
    w3 = W3.astype(jnp.bfloat16)

    full = lambda shape: pl.BlockSpec(shape, lambda b: (0,) * len(shape))
    out = pl.pallas_call(
        _deepset_body,
        grid=(B,),
        in_specs=[
            pl.BlockSpec((1, L, F), lambda b: (b, 0, 0)),
            full(w1.shape), full(s1.shape), full(b1.shape),
            full(w2.shape), full(s2.shape), full(b2.shape),
            full(w3.shape), full(s3.shape), full(b3.shape),
        ],
        out_specs=pl.BlockSpec((1, 1, 4 * H), lambda b: (b, 0, 0)),
        out_shape=jax.ShapeDtypeStruct((B, 1, 4 * H), jnp.float32),
        compiler_params=pltpu.CompilerParams(
            dimension_semantics=("parallel",)),
    )(inputs, w1, s1, b1, w2, s2, b2, w3, s3, b3)
    return out.reshape(B, 4 * H)


# speedup vs baseline: 1.0758x; 1.0529x over previous
import jax
import jax.numpy as jnp
import numpy as np
from jax.experimental import pallas as pl
from jax.experimental.pallas import tpu as pltpu

_BN_EPS = 1e-3


def _deepset_body(x_ref, w1_ref, b1_ref, w2_ref, b2_ref,
                  w3_ref, b3_ref, out_ref):
    x32 = x_ref[0]
    mask = jnp.any(x32 != 0.0, axis=1, keepdims=True)
    x = x32.astype(jnp.bfloat16)
    h = jnp.dot(x, w1_ref[...], preferred_element_type=jnp.float32)
    h = jnp.maximum(h.astype(jnp.bfloat16) + b1_ref[0], 0.0)
    h = jnp.dot(h, w2_ref[...], preferred_element_type=jnp.float32)
    h = jnp.maximum(h.astype(jnp.bfloat16) + b2_ref[0], 0.0)
    h = jnp.dot(h, w3_ref[...], preferred_element_type=jnp.float32)
    h = jnp.maximum(h + b3_ref[0], 0.0)

    hm = jnp.where(mask, h, 0.0)
    cnt = jnp.sum(mask.astype(jnp.float32))
    agg_sum = jnp.sum(hm, axis=0, keepdims=True)
    agg_max = jnp.max(hm, axis=0, keepdims=True)
    ex2 = jnp.sum(hm * hm, axis=0, keepdims=True) / cnt
    agg_mean = agg_sum / cnt
    var = ex2 - agg_mean * agg_mean
    agg_std = jnp.sqrt(jnp.maximum(var, 1e-12))

    H = agg_sum.shape[1]
    out_ref[0, :, 0 * H:1 * H] = agg_sum
    out_ref[0, :, 1 * H:2 * H] = agg_max
    out_ref[0, :, 2 * H:3 * H] = agg_mean
    out_ref[0, :, 3 * H:4 * H] = agg_std


def kernel(inputs, W1, g1, b1, W2, g2, b2, W3, g3, b3):
    B, L, F = inputs.shape
    H = W3.shape[1]
    inv = np.float32(1.0 / np.sqrt(1.0 + _BN_EPS))
    w1 = (W1 * (g1 * inv)[None, :]).astype(jnp.bfloat16)
    w2 = (W2 * (g2 * inv)[None, :]).astype(jnp.bfloat16)
    w3 = (W3 * (g3 * inv)[None, :]).astype(jnp.bfloat16)
    b1 = b1.reshape(1, -1).astype(jnp.bfloat16)
    b2 = b2.reshape(1, -1).astype(jnp.bfloat16)
    b3 = b3.reshape(1, -1)

    full = lambda shape: pl.BlockSpec(shape, lambda b: (0,) * len(shape))
    out = pl.pallas_call(
        _deepset_body,
        grid=(B,),
        in_specs=[
            pl.BlockSpec((1, L, F), lambda b: (b, 0, 0)),
            full(w1.shape), full(b1.shape),
            full(w2.shape), full(b2.shape),
            full(w3.shape), full(b3.shape),
        ],
        out_specs=pl.BlockSpec((1, 1, 4 * H), lambda b: (b, 0, 0)),
        out_shape=jax.ShapeDtypeStruct((B, 1, 4 * H), jnp.float32),
        compiler_params=pltpu.CompilerParams(
            dimension_semantics=("parallel",)),
    )(inputs, w1, b1, w2, b2, w3, b3)
    return out.reshape(B, 4 * H)


# speedup vs baseline: 1.1620x; 1.0802x over previous
import jax
import jax.numpy as jnp
import numpy as np
from jax.experimental import pallas as pl
from jax.experimental.pallas import tpu as pltpu

_BN_EPS = 1e-3


def _deepset_body(x_ref, w1_ref, b1_ref, w2_ref, b2_ref,
                  w3_ref, b3_ref, out_ref):
    L = x_ref.shape[1]
    n_chunks = 4
    C = L // n_chunks
    p_sum, p_max, p_sq, p_cnt = [], [], [], []

    def stage0(c):
        x32 = x_ref[0, c * C:(c + 1) * C, :]
        mask = jnp.any(x32 != 0.0, axis=1, keepdims=True)
        x = x32.astype(jnp.bfloat16)
        h = jnp.dot(x, w1_ref[...], preferred_element_type=jnp.float32)
        h = jnp.maximum(h.astype(jnp.bfloat16) + b1_ref[0], 0.0)
        return mask, h

    def stage1(h):
        h = jnp.dot(h, w2_ref[...], preferred_element_type=jnp.float32)
        return jnp.maximum(h.astype(jnp.bfloat16) + b2_ref[0], 0.0)

    def stage2(mask, h):
        h = jnp.dot(h, w3_ref[...], preferred_element_type=jnp.float32)
        hm = jnp.maximum(jnp.where(mask, h + b3_ref[0], -1.0), 0.0)
        p_cnt.append(jnp.sum(mask.astype(jnp.float32)))
        p_sum.append(jnp.sum(hm, axis=0, keepdims=True))
        p_max.append(jnp.max(hm, axis=0, keepdims=True))
        p_sq.append(jnp.sum(hm * hm, axis=0, keepdims=True))

    masks = [None] * n_chunks
    h1 = [None] * n_chunks
    h2 = [None] * n_chunks
    masks[0], h1[0] = stage0(0)
    masks[1], h1[1] = stage0(1)
    h2[0] = stage1(h1[0])
    for c in range(2, n_chunks):
        masks[c], h1[c] = stage0(c)
        h2[c - 1] = stage1(h1[c - 1])
        stage2(masks[c - 2], h2[c - 2])
    h2[n_chunks - 1] = stage1(h1[n_chunks - 1])
    stage2(masks[n_chunks - 2], h2[n_chunks - 2])
    stage2(masks[n_chunks - 1], h2[n_chunks - 1])

    cnt = sum(p_cnt)
    agg_sum = sum(p_sum)
    agg_max = p_max[0]
    for pm in p_max[1:]:
        agg_max = jnp.maximum(agg_max, pm)
    ex2 = sum(p_sq) / cnt
    agg_mean = agg_sum / cnt
    var = ex2 - agg_mean * agg_mean
    agg_std = jnp.sqrt(jnp.maximum(var, 1e-12))

    H = agg_sum.shape[1]
    out_ref[0, :, 0 * H:1 * H] = agg_sum
    out_ref[0, :, 1 * H:2 * H] = agg_max
    out_ref[0, :, 2 * H:3 * H] = agg_mean
    out_ref[0, :, 3 * H:4 * H] = agg_std


def kernel(inputs, W1, g1, b1, W2, g2, b2, W3, g3, b3):
    B, L, F = inputs.shape
    H = W3.shape[1]
    inv = np.float32(1.0 / np.sqrt(1.0 + _BN_EPS))
    w1 = (W1 * (g1 * inv)[None, :]).astype(jnp.bfloat16)
    w2 = (W2 * (g2 * inv)[None, :]).astype(jnp.bfloat16)
    w3 = (W3 * (g3 * inv)[None, :]).astype(jnp.bfloat16)
    b1 = b1.reshape(1, -1).astype(jnp.bfloat16)
    b2 = b2.reshape(1, -1).astype(jnp.bfloat16)
    b3 = b3.reshape(1, -1)

    full = lambda shape: pl.BlockSpec(shape, lambda b: (0,) * len(shape))
    out = pl.pallas_call(
        _deepset_body,
        grid=(B,),
        in_specs=[
            pl.BlockSpec((1, L, F), lambda b: (b, 0, 0)),
            full(w1.shape), full(b1.shape),
            full(w2.shape), full(b2.shape),
            full(w3.shape), full(b3.shape),
        ],
        out_specs=pl.BlockSpec((1, 1, 4 * H), lambda b: (b, 0, 0)),
        out_shape=jax.ShapeDtypeStruct((B, 1, 4 * H), jnp.float32),
        compiler_params=pltpu.CompilerParams(
            dimension_semantics=("parallel",)),
    )(inputs, w1, b1, w2, b2, w3, b3)
    return out.reshape(B, 4 * H)


# speedup vs baseline: 1.2962x; 1.1155x over previous
import jax
import jax.numpy as jnp
import numpy as np
from jax.experimental import pallas as pl
from jax.experimental.pallas import tpu as pltpu

_BN_EPS = 1e-3


def _deepset_body(x_ref, w1_ref, g1_ref, b1_ref, w2_ref, g2_ref, b2_ref,
                  w3_ref, g3_ref, b3_ref, out_ref,
                  w1s_ref, w2s_ref, w3s_ref, b12s_ref):
    inv = np.float32(1.0 / np.sqrt(1.0 + _BN_EPS))

    @pl.when(pl.program_id(0) == 0)
    def _prep():
        w1s_ref[...] = (w1_ref[...] * (g1_ref[0] * inv)).astype(jnp.bfloat16)
        w2s_ref[...] = (w2_ref[...] * (g2_ref[0] * inv)).astype(jnp.bfloat16)
        w3s_ref[...] = (w3_ref[...] * (g3_ref[0] * inv)).astype(jnp.bfloat16)
        b12s_ref[...] = jnp.concatenate(
            [b1_ref[...], b2_ref[...]], axis=0).astype(jnp.bfloat16)

    L = x_ref.shape[1]
    n_chunks = 4
    C = L // n_chunks
    b1 = b12s_ref[0:1]
    b2 = b12s_ref[1:2]
    b3 = b3_ref[...]
    p_sum, p_max, p_sq, p_cnt = [], [], [], []

    def stage0(c):
        x32 = x_ref[0, c * C:(c + 1) * C, :]
        mask = jnp.any(x32 != 0.0, axis=1, keepdims=True)
        x = x32.astype(jnp.bfloat16)
        h = jnp.dot(x, w1s_ref[...], preferred_element_type=jnp.float32)
        h = jnp.maximum(h.astype(jnp.bfloat16) + b1, 0.0)
        return mask, h

    def stage1(h):
        h = jnp.dot(h, w2s_ref[...], preferred_element_type=jnp.float32)
        return jnp.maximum(h.astype(jnp.bfloat16) + b2, 0.0)

    def stage2(mask, h):
        h = jnp.dot(h, w3s_ref[...], preferred_element_type=jnp.float32)
        hm = jnp.maximum(jnp.where(mask, h + b3, -1.0), 0.0)
        p_cnt.append(jnp.sum(mask.astype(jnp.float32)))
        p_sum.append(jnp.sum(hm, axis=0, keepdims=True))
        p_max.append(jnp.max(hm, axis=0, keepdims=True))
        p_sq.append(jnp.sum(hm * hm, axis=0, keepdims=True))

    masks = [None] * n_chunks
    h1 = [None] * n_chunks
    h2 = [None] * n_chunks
    masks[0], h1[0] = stage0(0)
    masks[1], h1[1] = stage0(1)
    h2[0] = stage1(h1[0])
    for c in range(2, n_chunks):
        masks[c], h1[c] = stage0(c)
        h2[c - 1] = stage1(h1[c - 1])
        stage2(masks[c - 2], h2[c - 2])
    h2[n_chunks - 1] = stage1(h1[n_chunks - 1])
    stage2(masks[n_chunks - 2], h2[n_chunks - 2])
    stage2(masks[n_chunks - 1], h2[n_chunks - 1])

    cnt = sum(p_cnt)
    agg_sum = sum(p_sum)
    agg_max = p_max[0]
    for pm in p_max[1:]:
        agg_max = jnp.maximum(agg_max, pm)
    ex2 = sum(p_sq) / cnt
    agg_mean = agg_sum / cnt
    var = ex2 - agg_mean * agg_mean
    agg_std = jnp.sqrt(jnp.maximum(var, 1e-12))

    H = agg_sum.shape[1]
    out_ref[0, :, 0 * H:1 * H] = agg_sum
    out_ref[0, :, 1 * H:2 * H] = agg_max
    out_ref[0, :, 2 * H:3 * H] = agg_mean
    out_ref[0, :, 3 * H:4 * H] = agg_std


def kernel(inputs, W1, g1, b1, W2, g2, b2, W3, g3, b3):
    B, L, F = inputs.shape
    H = W3.shape[1]

    full = lambda shape: pl.BlockSpec(shape, lambda b: (0,) * len(shape))
    vec = pl.BlockSpec((1, H), lambda b: (0, 0))
    out = pl.pallas_call(
        _deepset_body,
        grid=(B,),
        in_specs=[
            pl.BlockSpec((1, L, F), lambda b: (b, 0, 0)),
            full(W1.shape), vec, vec,
            full(W2.shape), vec, vec,
            full(W3.shape), vec, vec,
        ],
        out_specs=pl.BlockSpec((1, 1, 4 * H), lambda b: (b, 0, 0)),
        out_shape=jax.ShapeDtypeStruct((B, 1, 4 * H), jnp.float32),
        scratch_shapes=[
            pltpu.VMEM((F, H), jnp.bfloat16),
            pltpu.VMEM((H, H), jnp.bfloat16),
            pltpu.VMEM((H, H), jnp.bfloat16),
            pltpu.VMEM((2, H), jnp.bfloat16),
        ],
        compiler_params=pltpu.CompilerParams(
            dimension_semantics=("arbitrary",)),
    )(inputs, W1, g1.reshape(1, H), b1.reshape(1, H),
      W2, g2.reshape(1, H), b2.reshape(1, H),
      W3, g3.reshape(1, H), b3.reshape(1, H))
    return out.reshape(B, 4 * H)


# speedup vs baseline: 1.3260x; 1.0230x over previous
import jax
import jax.numpy as jnp
import numpy as np
from jax.experimental import pallas as pl
from jax.experimental.pallas import tpu as pltpu

_BN_EPS = 1e-3


def _deepset_body(x_ref, w1_ref, g1_ref, b1_ref, w2_ref, g2_ref, b2_ref,
                  w3_ref, g3_ref, b3_ref, out_ref,
                  w1s_ref, w2s_ref, w3s_ref, b12s_ref):
    inv = np.float32(1.0 / np.sqrt(1.0 + _BN_EPS))

    @pl.when(pl.program_id(0) == 0)
    def _prep():
        w1s_ref[...] = (w1_ref[...] * (g1_ref[...] * inv)[None, :]
                        ).astype(jnp.bfloat16)
        w2s_ref[...] = (w2_ref[...] * (g2_ref[...] * inv)[None, :]
                        ).astype(jnp.bfloat16)
        w3s_ref[...] = (w3_ref[...] * (g3_ref[...] * inv)[None, :]
                        ).astype(jnp.bfloat16)
        b12s_ref[...] = jnp.stack(
            [b1_ref[...], b2_ref[...]], axis=0).astype(jnp.bfloat16)

    L = x_ref.shape[1]
    n_chunks = 4
    C = L // n_chunks
    b1 = b12s_ref[0:1]
    b2 = b12s_ref[1:2]
    b3 = b3_ref[...][None, :]
    p_sum, p_max, p_sq, p_cnt = [], [], [], []

    def stage0(c):
        x32 = x_ref[0, c * C:(c + 1) * C, :]
        mask = jnp.any(x32 != 0.0, axis=1, keepdims=True)
        x = x32.astype(jnp.bfloat16)
        h = jnp.dot(x, w1s_ref[...], preferred_element_type=jnp.float32)
        h = jnp.maximum(h.astype(jnp.bfloat16) + b1, 0.0)
        return mask, h

    def stage1(h):
        h = jnp.dot(h, w2s_ref[...], preferred_element_type=jnp.float32)
        return jnp.maximum(h.astype(jnp.bfloat16) + b2, 0.0)

    def stage2(mask, h):
        h = jnp.dot(h, w3s_ref[...], preferred_element_type=jnp.float32)
        hm = jnp.maximum(jnp.where(mask, h + b3, -1.0), 0.0)
        p_cnt.append(jnp.sum(mask.astype(jnp.float32)))
        p_sum.append(jnp.sum(hm, axis=0, keepdims=True))
        p_max.append(jnp.max(hm, axis=0, keepdims=True))
        p_sq.append(jnp.sum(hm * hm, axis=0, keepdims=True))

    masks = [None] * n_chunks
    h1 = [None] * n_chunks
    h2 = [None] * n_chunks
    masks[0], h1[0] = stage0(0)
    masks[1], h1[1] = stage0(1)
    h2[0] = stage1(h1[0])
    for c in range(2, n_chunks):
        masks[c], h1[c] = stage0(c)
        h2[c - 1] = stage1(h1[c - 1])
        stage2(masks[c - 2], h2[c - 2])
    h2[n_chunks - 1] = stage1(h1[n_chunks - 1])
    stage2(masks[n_chunks - 2], h2[n_chunks - 2])
    stage2(masks[n_chunks - 1], h2[n_chunks - 1])

    cnt = sum(p_cnt)
    agg_sum = sum(p_sum)
    agg_max = p_max[0]
    for pm in p_max[1:]:
        agg_max = jnp.maximum(agg_max, pm)
    ex2 = sum(p_sq) / cnt
    agg_mean = agg_sum / cnt
    var = ex2 - agg_mean * agg_mean
    agg_std = jnp.sqrt(jnp.maximum(var, 1e-12))

    H = agg_sum.shape[1]
    b = pl.program_id(0)
    row = jnp.concatenate([agg_sum, agg_max, agg_mean, agg_std], axis=1)
    out_ref[pl.ds(b, 1), :] = row


def kernel(inputs, W1, g1, b1, W2, g2, b2, W3, g3, b3):
    B, L, F = inputs.shape
    H = W3.shape[1]

    full = lambda shape: pl.BlockSpec(shape, lambda b: (0,) * len(shape))
    return pl.pallas_call(
        _deepset_body,
        grid=(B,),
        in_specs=[
            pl.BlockSpec((1, L, F), lambda b: (b, 0, 0)),
            full(W1.shape), full(g1.shape), full(b1.shape),
            full(W2.shape), full(g2.shape), full(b2.shape),
            full(W3.shape), full(g3.shape), full(b3.shape),
        ],
        out_specs=pl.BlockSpec((B, 4 * H), lambda b: (0, 0)),
        out_shape=jax.ShapeDtypeStruct((B, 4 * H), jnp.float32),
        scratch_shapes=[
            pltpu.VMEM((F, H), jnp.bfloat16),
            pltpu.VMEM((H, H), jnp.bfloat16),
            pltpu.VMEM((H, H), jnp.bfloat16),
            pltpu.VMEM((2, H), jnp.bfloat16),
        ],
        compiler_params=pltpu.CompilerParams(
            dimension_semantics=("arbitrary",)),
    )(inputs, W1, g1, b1, W2, g2, b2, W3, g3, b3)


# speedup vs baseline: 1.4528x; 1.0956x over previous
import jax
import jax.numpy as jnp
import numpy as np
from jax.experimental import pallas as pl
from jax.experimental.pallas import tpu as pltpu

_BN_EPS = 1e-3


def _deepset_body(x_ref, w1_ref, g1_ref, b1_ref, w2_ref, g2_ref, b2_ref,
                  w3_ref, g3_ref, b3_ref, out_ref,
                  w1s_ref, w2s_ref, w3s_ref, b12s_ref):
    inv = np.float32(1.0 / np.sqrt(1.0 + _BN_EPS))

    @pl.when(pl.program_id(0) == 0)
    def _prep():
        w1s_ref[...] = (w1_ref[...] * (g1_ref[...] * inv)[None, :]
                        ).astype(jnp.bfloat16)
        w2s_ref[...] = (w2_ref[...] * (g2_ref[...] * inv)[None, :]
                        ).astype(jnp.bfloat16)
        w3s_ref[...] = (w3_ref[...] * (g3_ref[...] * inv)[None, :]
                        ).astype(jnp.bfloat16)
        b12s_ref[...] = jnp.stack(
            [b1_ref[...], b2_ref[...]], axis=0).astype(jnp.bfloat16)

    L = x_ref.shape[2]
    n_chunks = 4
    C = L // n_chunks
    b1 = b12s_ref[0:1]
    b2 = b12s_ref[1:2]
    b3 = b3_ref[...][None, :]
    p_sum, p_max, p_sq, p_cnt = [], [], [], []

    def stage0(c):
        xt = x_ref[0, :, c * C:(c + 1) * C]
        mask = jnp.any(xt != 0.0, axis=0, keepdims=True)
        mask = jnp.transpose(mask)
        x = xt.astype(jnp.bfloat16)
        h = jax.lax.dot_general(x, w1s_ref[...], (((0,), (0,)), ((), ())),
                                preferred_element_type=jnp.float32)
        h = jnp.maximum(h.astype(jnp.bfloat16) + b1, 0.0)
        return mask, h

    def stage1(h):
        h = jnp.dot(h, w2s_ref[...], preferred_element_type=jnp.float32)
        return jnp.maximum(h.astype(jnp.bfloat16) + b2, 0.0)

    def stage2(mask, h):
        h = jnp.dot(h, w3s_ref[...], preferred_element_type=jnp.float32)
        hm = jnp.maximum(jnp.where(mask, h + b3, -1.0), 0.0)
        p_cnt.append(jnp.sum(mask.astype(jnp.float32)))
        p_sum.append(jnp.sum(hm, axis=0, keepdims=True))
        p_max.append(jnp.max(hm, axis=0, keepdims=True))
        p_sq.append(jnp.sum(hm * hm, axis=0, keepdims=True))

    masks = [None] * n_chunks
    h1 = [None] * n_chunks
    h2 = [None] * n_chunks
    masks[0], h1[0] = stage0(0)
    masks[1], h1[1] = stage0(1)
    h2[0] = stage1(h1[0])
    for c in range(2, n_chunks):
        masks[c], h1[c] = stage0(c)
        h2[c - 1] = stage1(h1[c - 1])
        stage2(masks[c - 2], h2[c - 2])
    h2[n_chunks - 1] = stage1(h1[n_chunks - 1])
    stage2(masks[n_chunks - 2], h2[n_chunks - 2])
    stage2(masks[n_chunks - 1], h2[n_chunks - 1])

    cnt = sum(p_cnt)
    agg_sum = sum(p_sum)
    agg_max = p_max[0]
    for pm in p_max[1:]:
        agg_max = jnp.maximum(agg_max, pm)
    ex2 = sum(p_sq) / cnt
    agg_mean = agg_sum / cnt
    var = ex2 - agg_mean * agg_mean
    agg_std = jnp.sqrt(jnp.maximum(var, 1e-12))

    H = agg_sum.shape[1]
    b = pl.program_id(0)
    row = jnp.concatenate([agg_sum, agg_max, agg_mean, agg_std], axis=1)
    out_ref[pl.ds(b, 1), :] = row


def kernel(inputs, W1, g1, b1, W2, g2, b2, W3, g3, b3):
    B, L, F = inputs.shape
    H = W3.shape[1]

    full = lambda shape: pl.BlockSpec(shape, lambda b: (0,) * len(shape))
    return pl.pallas_call(
        _deepset_body,
        grid=(B,),
        in_specs=[
            pl.BlockSpec((1, F, L), lambda b: (b, 0, 0)),
            full(W1.shape), full(g1.shape), full(b1.shape),
            full(W2.shape), full(g2.shape), full(b2.shape),
            full(W3.shape), full(g3.shape), full(b3.shape),
        ],
        out_specs=pl.BlockSpec((B, 4 * H), lambda b: (0, 0)),
        out_shape=jax.ShapeDtypeStruct((B, 4 * H), jnp.float32),
        scratch_shapes=[
            pltpu.VMEM((F, H), jnp.bfloat16),
            pltpu.VMEM((H, H), jnp.bfloat16),
            pltpu.VMEM((H, H), jnp.bfloat16),
            pltpu.VMEM((2, H), jnp.bfloat16),
        ],
        compiler_params=pltpu.CompilerParams(
            dimension_semantics=("arbitrary",)),
    )(inputs.transpose(0, 2, 1), W1, g1, b1, W2, g2, b2, W3, g3, b3)


# speedup vs baseline: 1.7677x; 1.2168x over previous
import jax
import jax.numpy as jnp
import numpy as np
from jax.experimental import pallas as pl
from jax.experimental.pallas import tpu as pltpu

_BN_EPS = 1e-3


def _deepset_body(x_ref, w1_ref, g1_ref, b1_ref, w2_ref, g2_ref, b2_ref,
                  w3_ref, g3_ref, b3_ref, out_ref,
                  w1s_ref, w2s_ref, w3s_ref, b12s_ref):
    inv = np.float32(1.0 / np.sqrt(1.0 + _BN_EPS))

    @pl.when(pl.program_id(0) == 0)
    def _prep():
        w1s_ref[...] = (w1_ref[...] * (g1_ref[...] * inv)[None, :]
                        ).astype(jnp.bfloat16)
        w2s_ref[...] = (w2_ref[...] * (g2_ref[...] * inv)[None, :]
                        ).astype(jnp.bfloat16)
        w3s_ref[...] = (w3_ref[...] * (g3_ref[...] * inv)[None, :]
                        ).astype(jnp.bfloat16)
        b12s_ref[...] = jnp.stack(
            [b1_ref[...], b2_ref[...]], axis=0).astype(jnp.bfloat16)

    L = x_ref.shape[2]
    n_chunks = 4
    C = L // n_chunks
    R = x_ref.shape[0]
    b1 = b12s_ref[0:1]
    b2 = b12s_ref[1:2]
    b3 = b3_ref[...][None, :]
    acc = [dict(p_sum=[], p_max=[], p_sq=[], p_cnt=[]) for _ in range(R)]

    def stage0(it):
        r, c = it
        xt = x_ref[r, :, c * C:(c + 1) * C]
        xc = jnp.transpose(xt)
        mask = jnp.any(xc != 0.0, axis=1, keepdims=True)
        x = xt.astype(jnp.bfloat16)
        h = jax.lax.dot_general(x, w1s_ref[...], (((0,), (0,)), ((), ())),
                                preferred_element_type=jnp.float32)
        h = jnp.maximum(h.astype(jnp.bfloat16) + b1, 0.0)
        return mask, h

    def stage1(h):
        h = jnp.dot(h, w2s_ref[...], preferred_element_type=jnp.float32)
        return jnp.maximum(h.astype(jnp.bfloat16) + b2, 0.0)

    def stage2(mask, h):
        h = jnp.dot(h, w3s_ref[...], preferred_element_type=jnp.float32)
        return jnp.maximum((h + b3) * mask.astype(jnp.float32), 0.0)

    def stage3(it, mask, hm):
        a = acc[it[0]]
        a["p_cnt"].append(jnp.sum(mask.astype(jnp.float32)))
        a["p_sum"].append(jnp.sum(hm, axis=0, keepdims=True))

    def stage4(it, hm):
        a = acc[it[0]]
        a["p_max"].append(jnp.max(hm, axis=0, keepdims=True))
        a["p_sq"].append(jnp.sum(hm * hm, axis=0, keepdims=True))

    items = [(r, c) for c in range(n_chunks) for r in range(R)]
    n = len(items)
    masks, h1, h2, hm = {}, {}, {}, {}

    def run(stage, i):
        if i < 0 or i >= n:
            return
        it = items[i]
        if stage == 0:
            masks[it], h1[it] = stage0(it)
        elif stage == 1:
            h2[it] = stage1(h1[it])
        elif stage == 2:
            hm[it] = stage2(masks[it], h2[it])
        elif stage == 3:
            stage3(it, masks[it], hm[it])
        else:
            stage4(it, hm[it])

    for i in range(n + 4):
        run(0, i)
        run(1, i - 1)
        run(2, i - 2)
        run(3, i - 3)
        run(4, i - 4)

    for r in range(R):
        a = acc[r]
        cnt = sum(a["p_cnt"])
        agg_sum = sum(a["p_sum"])
        agg_max = a["p_max"][0]
        for pm in a["p_max"][1:]:
            agg_max = jnp.maximum(agg_max, pm)
        ex2 = sum(a["p_sq"]) / cnt
        agg_mean = agg_sum / cnt
        var = ex2 - agg_mean * agg_mean
        agg_std = jnp.sqrt(jnp.maximum(var, 1e-12))
        H = agg_sum.shape[1]
        b = pl.program_id(0) * R + r
        row = jnp.concatenate([agg_sum, agg_max, agg_mean, agg_std], axis=1)
        out_ref[pl.ds(b, 1), :] = row


def kernel(inputs, W1, g1, b1, W2, g2, b2, W3, g3, b3):
    B, L, F = inputs.shape
    H = W3.shape[1]

    full = lambda shape: pl.BlockSpec(shape, lambda b: (0,) * len(shape))
    return pl.pallas_call(
        _deepset_body,
        grid=(B // 2,),
        in_specs=[
            pl.BlockSpec((2, F, L), lambda b: (b, 0, 0)),
            full(W1.shape), full(g1.shape), full(b1.shape),
            full(W2.shape), full(g2.shape), full(b2.shape),
            full(W3.shape), full(g3.shape), full(b3.shape),
        ],
        out_specs=pl.BlockSpec((B, 4 * H), lambda b: (0, 0)),
        out_shape=jax.ShapeDtypeStruct((B, 4 * H), jnp.float32),
        scratch_shapes=[
            pltpu.VMEM((F, H), jnp.bfloat16),
            pltpu.VMEM((H, H), jnp.bfloat16),
            pltpu.VMEM((H, H), jnp.bfloat16),
            pltpu.VMEM((2, H), jnp.bfloat16),
        ],
        compiler_params=pltpu.CompilerParams(
            dimension_semantics=("arbitrary",)),
    )(inputs.transpose(0, 2, 1), W1, g1, b1, W2, g2, b2, W3, g3, b3)
